# Initial kernel scaffold; baseline (speedup 1.0000x reference)
import jax
import jax.numpy as jnp
from jax.experimental import pallas as pl


def kernel(input_ids, position_ids, token_table, pos_table):
    raise NotImplementedError("write your pallas kernel here")


# speedup vs baseline: 1.2618x; 1.2618x over previous
import functools

import jax
import jax.numpy as jnp
from jax import lax
from jax.experimental import pallas as pl
from jax.experimental.pallas import tpu as pltpu
from jax.experimental.pallas import tpu_sc as plsc

NC = 2
NS = 16
NW = NC * NS
L = 16


def _make_kernel(N, D, K):
    assert N % (NW * K) == 0 and D % L == 0
    b_per_w = N // NW
    steps = b_per_w // K
    mesh = plsc.VectorSubcoreMesh(core_axis_name="c", subcore_axis_name="s")

    @functools.partial(
        pl.kernel,
        mesh=mesh,
        out_type=jax.ShapeDtypeStruct((N, D), jnp.float32),
        scratch_types=[
            pltpu.VMEM((K,), jnp.int32),
            pltpu.VMEM((K,), jnp.int32),
            pltpu.VMEM((K, D), jnp.float32),
            pltpu.VMEM((K, D), jnp.float32),
            pltpu.SemaphoreType.DMA,
            pltpu.SemaphoreType.DMA,
        ],
    )
    def emb_kernel(ids_hbm, pids_hbm, tok_hbm, pos_hbm, out_hbm,
                   idx_v, pidx_v, tok_b, pos_b, sem_t, sem_p):
        wid = lax.axis_index("s") * NC + lax.axis_index("c")
        base = wid * b_per_w

        def step(i, carry):
            off = base + i * K
            pltpu.sync_copy(ids_hbm.at[pl.ds(off, K)], idx_v)
            pltpu.sync_copy(pids_hbm.at[pl.ds(off, K)], pidx_v)
            cp_t = pltpu.async_copy(tok_hbm.at[idx_v], tok_b, sem_t)
            cp_p = pltpu.async_copy(pos_hbm.at[pidx_v], pos_b, sem_p)
            cp_t.wait()
            cp_p.wait()

            def add_row(r, c):
                for d in range(D // L):
                    sl = pl.ds(d * L, L)
                    tok_b[r, sl] = tok_b[r, sl] + pos_b[r, sl]
                return c

            lax.fori_loop(0, K, add_row, 0)
            pltpu.sync_copy(tok_b, out_hbm.at[pl.ds(off, K)])
            return carry

        lax.fori_loop(0, steps, step, 0)

    return emb_kernel


def kernel(input_ids, position_ids, token_table, pos_table):
    B, S = input_ids.shape
    V, D = token_table.shape
    N = B * S
    ids = input_ids.reshape(N).astype(jnp.int32)
    pids = position_ids.reshape(N).astype(jnp.int32)
    k = _make_kernel(N, D, K=64)
    out = k(ids, pids, token_table, pos_table)
    return out.reshape(B, S, D)
